# baseline (speedup 1.0000x reference)
import functools

import jax
import jax.numpy as jnp
from jax.experimental import pallas as pl
from jax.experimental.pallas import tpu as pltpu

OLD_CL = 16
RATIO = 0.2
BH = 64


def _fold(t):
    bh, w = t.shape
    return t.reshape(bh // 8, 8, w // 128, 128).sum(axis=(0, 2))


def _loss_kernel(x_ref, out_ref, vec_ref, img_ref, *, n_img, n_j, c, h, w):
    i = pl.program_id(0)
    j = pl.program_id(1)
    n_new = c - OLD_CL
    nq = 2 * n_new + 1

    @pl.when(j == 0)
    def _init_vec():
        vec_ref[:, :, :] = jnp.zeros_like(vec_ref)

    @pl.when((i == 0) & (j == 0))
    def _init_img():
        img_ref[:, :] = jnp.zeros_like(img_ref)

    x = x_ref[0]

    m = x[0]
    for ci in range(1, c):
        m = jnp.maximum(m, x[ci])

    s_old = jnp.exp(x[0] - m)
    for ci in range(1, OLD_CL):
        s_old = s_old + jnp.exp(x[ci] - m)
    e_new = [jnp.exp(x[ci] - m) for ci in range(OLD_CL, c)]
    z = s_old
    for e in e_new:
        z = z + e
    inv = 1.0 / z

    p0 = s_old * inv
    upd = [_fold(p0 * p0)]
    for e in e_new:
        p = e * inv
        upd.append(_fold(p * p))
    for ci in range(OLD_CL, c):
        upd.append(_fold(jnp.where(x[ci] == m, 1.0, 0.0)))

    for k in range(nq):
        vec_ref[k, :, :] = vec_ref[k, :, :] + upd[k]

    @pl.when(j == n_j - 1)
    def _flush():
        row = jax.lax.broadcasted_iota(jnp.int32, (8, 128), 0)
        lane = jax.lax.broadcasted_iota(jnp.int32, (8, 128), 1)
        scalars = [jnp.sum(vec_ref[k, :, :]) for k in range(nq)]
        cnt_new = scalars[1 + n_new:]
        cnt_old = float(h * w) - sum(cnt_new)
        vals = scalars[: 1 + n_new] + [cnt_old] + cnt_new
        acc = img_ref[:, :]
        for k, v in enumerate(vals):
            acc = acc + jnp.where((row == i) & (lane == k), v, 0.0)
        img_ref[:, :] = acc

        @pl.when(i == n_img - 1)
        def _finish():
            nbin = n_new + 1
            sq_lane = lane < nbin
            cnt_lane = (lane >= nbin) & (lane < 2 * nbin)
            valid = row < n_img
            a = img_ref[:, :]
            cnt = jnp.where(valid & cnt_lane, a, 0.0)
            safe = jnp.where(valid & cnt_lane,
                             jnp.where(cnt == 0.0, 1.0, cnt), 1.0)
            total = jnp.sum(jnp.where(valid & cnt_lane, safe, 0.0),
                            axis=1, keepdims=True)
            wgt = jnp.where(valid & cnt_lane,
                            jnp.power(total / safe, RATIO), 0.0)
            sq = jnp.where(valid & sq_lane, a, 0.0)
            contrib = jnp.sum(sq * jnp.roll(wgt, -nbin, axis=1))
            out_ref[0, 0] = -contrib / (n_img * c * h * w)


def kernel(inputs):
    n, c, h, w = inputs.shape
    n_j = h // BH
    nq = 2 * (c - OLD_CL) + 1
    out = pl.pallas_call(
        functools.partial(_loss_kernel, n_img=n, n_j=n_j, c=c, h=h, w=w),
        grid=(n, n_j),
        in_specs=[
            pl.BlockSpec((1, c, BH, w), lambda i, j: (i, 0, j, 0)),
        ],
        out_specs=pl.BlockSpec(
            (1, 1), lambda i, j: (0, 0), memory_space=pltpu.SMEM
        ),
        out_shape=jax.ShapeDtypeStruct((1, 1), jnp.float32),
        scratch_shapes=[
            pltpu.VMEM((nq, 8, 128), jnp.float32),
            pltpu.VMEM((8, 128), jnp.float32),
        ],
    )(inputs)
    return out[0, 0]


# speedup vs baseline: 1.2847x; 1.2847x over previous
import functools

import jax
import jax.numpy as jnp
from jax.experimental import pallas as pl
from jax.experimental.pallas import tpu as pltpu

OLD_CL = 16
RATIO = 0.2
BH = 64


def _fold(t):
    bh, w = t.shape
    return t.reshape(bh // 8, 8, w).sum(axis=0)


def _loss_kernel(x_ref, out_ref, vec_ref, img_ref, *, n_img, n_j, c, h, w):
    i = pl.program_id(0)
    j = pl.program_id(1)
    n_new = c - OLD_CL
    nq = 2 * n_new + 1

    @pl.when(j == 0)
    def _init_vec():
        vec_ref[:, :, :] = jnp.zeros_like(vec_ref)

    @pl.when((i == 0) & (j == 0))
    def _init_img():
        img_ref[:, :] = jnp.zeros_like(img_ref)

    x = x_ref[0]

    e0 = jnp.exp(x[0])
    m = e0
    s_old = e0
    for ci in range(1, OLD_CL):
        e = jnp.exp(x[ci])
        m = jnp.maximum(m, e)
        s_old = s_old + e
    e_new = []
    z = s_old
    for ci in range(OLD_CL, c):
        e = jnp.exp(x[ci])
        m = jnp.maximum(m, e)
        z = z + e
        e_new.append(e)
    inv = 1.0 / z

    p0 = s_old * inv
    upd = [_fold(p0 * p0)]
    for e in e_new:
        p = e * inv
        upd.append(_fold(p * p))
    for e in e_new:
        upd.append(_fold(jnp.where(e == m, 1.0, 0.0)))

    for k in range(nq):
        vec_ref[k, :, :] = vec_ref[k, :, :] + upd[k]

    @pl.when(j == n_j - 1)
    def _flush():
        row = jax.lax.broadcasted_iota(jnp.int32, (8, 128), 0)
        lane = jax.lax.broadcasted_iota(jnp.int32, (8, 128), 1)
        scalars = [jnp.sum(vec_ref[k, :, :]) for k in range(nq)]
        cnt_new = scalars[1 + n_new:]
        cnt_old = float(h * w) - sum(cnt_new)
        vals = scalars[: 1 + n_new] + [cnt_old] + cnt_new
        acc = img_ref[:, :]
        for k, v in enumerate(vals):
            acc = acc + jnp.where((row == i) & (lane == k), v, 0.0)
        img_ref[:, :] = acc

        @pl.when(i == n_img - 1)
        def _finish():
            nbin = n_new + 1
            sq_lane = lane < nbin
            cnt_lane = (lane >= nbin) & (lane < 2 * nbin)
            valid = row < n_img
            a = img_ref[:, :]
            cnt = jnp.where(valid & cnt_lane, a, 0.0)
            safe = jnp.where(valid & cnt_lane,
                             jnp.where(cnt == 0.0, 1.0, cnt), 1.0)
            total = jnp.sum(jnp.where(valid & cnt_lane, safe, 0.0),
                            axis=1, keepdims=True)
            wgt = jnp.where(valid & cnt_lane,
                            jnp.power(total / safe, RATIO), 0.0)
            sq = jnp.where(valid & sq_lane, a, 0.0)
            contrib = jnp.sum(sq * jnp.roll(wgt, -nbin, axis=1))
            out_ref[0, 0] = -contrib / (n_img * c * h * w)


def kernel(inputs):
    n, c, h, w = inputs.shape
    n_j = h // BH
    nq = 2 * (c - OLD_CL) + 1
    out = pl.pallas_call(
        functools.partial(_loss_kernel, n_img=n, n_j=n_j, c=c, h=h, w=w),
        grid=(n, n_j),
        in_specs=[
            pl.BlockSpec((1, c, BH, w), lambda i, j: (i, 0, j, 0)),
        ],
        out_specs=pl.BlockSpec(
            (1, 1), lambda i, j: (0, 0), memory_space=pltpu.SMEM
        ),
        out_shape=jax.ShapeDtypeStruct((1, 1), jnp.float32),
        scratch_shapes=[
            pltpu.VMEM((nq, 8, w), jnp.float32),
            pltpu.VMEM((8, 128), jnp.float32),
        ],
    )(inputs)
    return out[0, 0]


# speedup vs baseline: 1.5261x; 1.1879x over previous
import functools

import jax
import jax.numpy as jnp
from jax.experimental import pallas as pl
from jax.experimental.pallas import tpu as pltpu

OLD_CL = 16
RATIO = 0.2
BH = 128


def _fold(t):
    bh, w = t.shape
    return t.reshape(bh // 8, 8, w).sum(axis=0)


def _loss_kernel(x_ref, out_ref, vec_ref, img_ref, *, n_img, n_j, c, h, w):
    i = pl.program_id(0)
    j = pl.program_id(1)
    n_new = c - OLD_CL
    nq = 2 * n_new + 1

    @pl.when(j == 0)
    def _init_vec():
        vec_ref[:, :, :] = jnp.zeros_like(vec_ref)

    @pl.when((i == 0) & (j == 0))
    def _init_img():
        img_ref[:, :] = jnp.zeros_like(img_ref)

    x = x_ref[0]

    e0 = jnp.exp(x[0])
    m = e0
    s_old = e0
    for ci in range(1, OLD_CL):
        e = jnp.exp(x[ci])
        m = jnp.maximum(m, e)
        s_old = s_old + e
    e_new = []
    z = s_old
    for ci in range(OLD_CL, c):
        e = jnp.exp(x[ci])
        m = jnp.maximum(m, e)
        z = z + e
        e_new.append(e)
    inv = 1.0 / z

    p0 = s_old * inv
    upd = [_fold(p0 * p0)]
    for e in e_new:
        p = e * inv
        upd.append(_fold(p * p))
    for e in e_new:
        upd.append(_fold(jnp.where(e == m, 1.0, 0.0)))

    for k in range(nq):
        vec_ref[k, :, :] = vec_ref[k, :, :] + upd[k]

    @pl.when(j == n_j - 1)
    def _flush():
        row = jax.lax.broadcasted_iota(jnp.int32, (8, 128), 0)
        lane = jax.lax.broadcasted_iota(jnp.int32, (8, 128), 1)
        scalars = [jnp.sum(vec_ref[k, :, :]) for k in range(nq)]
        cnt_new = scalars[1 + n_new:]
        cnt_old = float(h * w) - sum(cnt_new)
        vals = scalars[: 1 + n_new] + [cnt_old] + cnt_new
        acc = img_ref[:, :]
        for k, v in enumerate(vals):
            acc = acc + jnp.where((row == i) & (lane == k), v, 0.0)
        img_ref[:, :] = acc

        @pl.when(i == n_img - 1)
        def _finish():
            nbin = n_new + 1
            sq_lane = lane < nbin
            cnt_lane = (lane >= nbin) & (lane < 2 * nbin)
            valid = row < n_img
            a = img_ref[:, :]
            cnt = jnp.where(valid & cnt_lane, a, 0.0)
            safe = jnp.where(valid & cnt_lane,
                             jnp.where(cnt == 0.0, 1.0, cnt), 1.0)
            total = jnp.sum(jnp.where(valid & cnt_lane, safe, 0.0),
                            axis=1, keepdims=True)
            wgt = jnp.where(valid & cnt_lane,
                            jnp.power(total / safe, RATIO), 0.0)
            sq = jnp.where(valid & sq_lane, a, 0.0)
            contrib = jnp.sum(sq * jnp.roll(wgt, -nbin, axis=1))
            out_ref[0, 0] = -contrib / (n_img * c * h * w)


def kernel(inputs):
    n, c, h, w = inputs.shape
    n_j = h // BH
    nq = 2 * (c - OLD_CL) + 1
    out = pl.pallas_call(
        functools.partial(_loss_kernel, n_img=n, n_j=n_j, c=c, h=h, w=w),
        grid=(n, n_j),
        in_specs=[
            pl.BlockSpec((1, c, BH, w), lambda i, j: (i, 0, j, 0)),
        ],
        out_specs=pl.BlockSpec(
            (1, 1), lambda i, j: (0, 0), memory_space=pltpu.SMEM
        ),
        out_shape=jax.ShapeDtypeStruct((1, 1), jnp.float32),
        scratch_shapes=[
            pltpu.VMEM((nq, 8, w), jnp.float32),
            pltpu.VMEM((8, 128), jnp.float32),
        ],
    )(inputs)
    return out[0, 0]


# speedup vs baseline: 1.5979x; 1.0470x over previous
import functools

import jax
import jax.numpy as jnp
from jax.experimental import pallas as pl
from jax.experimental.pallas import tpu as pltpu

OLD_CL = 16
RATIO = 0.2
BH = 256


def _fold(t):
    bh, w = t.shape
    return t.reshape(bh // 8, 8, w).sum(axis=0)


def _loss_kernel(x_ref, out_ref, vec_ref, img_ref, *, n_img, n_j, c, h, w):
    i = pl.program_id(0)
    j = pl.program_id(1)
    n_new = c - OLD_CL
    nq = 2 * n_new + 1

    @pl.when(j == 0)
    def _init_vec():
        vec_ref[:, :, :] = jnp.zeros_like(vec_ref)

    @pl.when((i == 0) & (j == 0))
    def _init_img():
        img_ref[:, :] = jnp.zeros_like(img_ref)

    x = x_ref[0]

    e0 = jnp.exp(x[0])
    m = e0
    s_old = e0
    for ci in range(1, OLD_CL):
        e = jnp.exp(x[ci])
        m = jnp.maximum(m, e)
        s_old = s_old + e
    e_new = []
    z = s_old
    for ci in range(OLD_CL, c):
        e = jnp.exp(x[ci])
        m = jnp.maximum(m, e)
        z = z + e
        e_new.append(e)
    inv = 1.0 / z

    p0 = s_old * inv
    upd = [_fold(p0 * p0)]
    for e in e_new:
        p = e * inv
        upd.append(_fold(p * p))
    for e in e_new:
        upd.append(_fold(jnp.where(e == m, 1.0, 0.0)))

    for k in range(nq):
        vec_ref[k, :, :] = vec_ref[k, :, :] + upd[k]

    @pl.when(j == n_j - 1)
    def _flush():
        row = jax.lax.broadcasted_iota(jnp.int32, (8, 128), 0)
        lane = jax.lax.broadcasted_iota(jnp.int32, (8, 128), 1)
        scalars = [jnp.sum(vec_ref[k, :, :]) for k in range(nq)]
        cnt_new = scalars[1 + n_new:]
        cnt_old = float(h * w) - sum(cnt_new)
        vals = scalars[: 1 + n_new] + [cnt_old] + cnt_new
        acc = img_ref[:, :]
        for k, v in enumerate(vals):
            acc = acc + jnp.where((row == i) & (lane == k), v, 0.0)
        img_ref[:, :] = acc

        @pl.when(i == n_img - 1)
        def _finish():
            nbin = n_new + 1
            sq_lane = lane < nbin
            cnt_lane = (lane >= nbin) & (lane < 2 * nbin)
            valid = row < n_img
            a = img_ref[:, :]
            cnt = jnp.where(valid & cnt_lane, a, 0.0)
            safe = jnp.where(valid & cnt_lane,
                             jnp.where(cnt == 0.0, 1.0, cnt), 1.0)
            total = jnp.sum(jnp.where(valid & cnt_lane, safe, 0.0),
                            axis=1, keepdims=True)
            wgt = jnp.where(valid & cnt_lane,
                            jnp.power(total / safe, RATIO), 0.0)
            sq = jnp.where(valid & sq_lane, a, 0.0)
            contrib = jnp.sum(sq * jnp.roll(wgt, -nbin, axis=1))
            out_ref[0, 0] = -contrib / (n_img * c * h * w)


def kernel(inputs):
    n, c, h, w = inputs.shape
    n_j = h // BH
    nq = 2 * (c - OLD_CL) + 1
    out = pl.pallas_call(
        functools.partial(_loss_kernel, n_img=n, n_j=n_j, c=c, h=h, w=w),
        grid=(n, n_j),
        in_specs=[
            pl.BlockSpec((1, c, BH, w), lambda i, j: (i, 0, j, 0)),
        ],
        out_specs=pl.BlockSpec(
            (1, 1), lambda i, j: (0, 0), memory_space=pltpu.SMEM
        ),
        out_shape=jax.ShapeDtypeStruct((1, 1), jnp.float32),
        scratch_shapes=[
            pltpu.VMEM((nq, 8, w), jnp.float32),
            pltpu.VMEM((8, 128), jnp.float32),
        ],
    )(inputs)
    return out[0, 0]


# speedup vs baseline: 1.7148x; 1.0732x over previous
import functools

import jax
import jax.numpy as jnp
from jax.experimental import pallas as pl
from jax.experimental.pallas import tpu as pltpu

OLD_CL = 16
RATIO = 0.2
BH = 256


SR = 8


def _loss_kernel(x_ref, out_ref, vec_ref, img_ref, *, n_img, n_j, c, h, w):
    i = pl.program_id(0)
    j = pl.program_id(1)
    n_new = c - OLD_CL
    nq = 2 * n_new + 1

    @pl.when(j == 0)
    def _init_vec():
        vec_ref[:, :, :] = jnp.zeros_like(vec_ref)

    @pl.when((i == 0) & (j == 0))
    def _init_img():
        img_ref[:, :] = jnp.zeros_like(img_ref)

    accs = [jnp.zeros((SR, w), dtype=jnp.float32) for _ in range(nq)]
    for s in range(0, x_ref.shape[2], SR):
        sl = pl.ds(s, SR)
        e0 = jnp.exp(x_ref[0, 0, sl, :])
        m = e0
        s_old = e0
        for ci in range(1, OLD_CL):
            e = jnp.exp(x_ref[0, ci, sl, :])
            m = jnp.maximum(m, e)
            s_old = s_old + e
        e_new = []
        z = s_old
        for ci in range(OLD_CL, c):
            e = jnp.exp(x_ref[0, ci, sl, :])
            m = jnp.maximum(m, e)
            z = z + e
            e_new.append(e)
        inv = 1.0 / z

        p0 = s_old * inv
        accs[0] = accs[0] + p0 * p0
        for k, e in enumerate(e_new):
            p = e * inv
            accs[1 + k] = accs[1 + k] + p * p
        for k, e in enumerate(e_new):
            accs[1 + n_new + k] = accs[1 + n_new + k] + jnp.where(
                e == m, 1.0, 0.0)

    for k in range(nq):
        vec_ref[k, :, :] = vec_ref[k, :, :] + accs[k]

    @pl.when(j == n_j - 1)
    def _flush():
        row = jax.lax.broadcasted_iota(jnp.int32, (8, 128), 0)
        lane = jax.lax.broadcasted_iota(jnp.int32, (8, 128), 1)
        scalars = [jnp.sum(vec_ref[k, :, :]) for k in range(nq)]
        cnt_new = scalars[1 + n_new:]
        cnt_old = float(h * w) - sum(cnt_new)
        vals = scalars[: 1 + n_new] + [cnt_old] + cnt_new
        acc = img_ref[:, :]
        for k, v in enumerate(vals):
            acc = acc + jnp.where((row == i) & (lane == k), v, 0.0)
        img_ref[:, :] = acc

        @pl.when(i == n_img - 1)
        def _finish():
            nbin = n_new + 1
            sq_lane = lane < nbin
            cnt_lane = (lane >= nbin) & (lane < 2 * nbin)
            valid = row < n_img
            a = img_ref[:, :]
            cnt = jnp.where(valid & cnt_lane, a, 0.0)
            safe = jnp.where(valid & cnt_lane,
                             jnp.where(cnt == 0.0, 1.0, cnt), 1.0)
            total = jnp.sum(jnp.where(valid & cnt_lane, safe, 0.0),
                            axis=1, keepdims=True)
            wgt = jnp.where(valid & cnt_lane,
                            jnp.power(total / safe, RATIO), 0.0)
            sq = jnp.where(valid & sq_lane, a, 0.0)
            contrib = jnp.sum(sq * jnp.roll(wgt, -nbin, axis=1))
            out_ref[0, 0] = -contrib / (n_img * c * h * w)


def kernel(inputs):
    n, c, h, w = inputs.shape
    n_j = h // BH
    nq = 2 * (c - OLD_CL) + 1
    out = pl.pallas_call(
        functools.partial(_loss_kernel, n_img=n, n_j=n_j, c=c, h=h, w=w),
        grid=(n, n_j),
        in_specs=[
            pl.BlockSpec((1, c, BH, w), lambda i, j: (i, 0, j, 0)),
        ],
        out_specs=pl.BlockSpec(
            (1, 1), lambda i, j: (0, 0), memory_space=pltpu.SMEM
        ),
        out_shape=jax.ShapeDtypeStruct((1, 1), jnp.float32),
        scratch_shapes=[
            pltpu.VMEM((nq, 8, w), jnp.float32),
            pltpu.VMEM((8, 128), jnp.float32),
        ],
    )(inputs)
    return out[0, 0]
